# baseline (speedup 1.0000x reference)
import functools

import jax
import jax.numpy as jnp
from jax import lax
from jax.experimental import pallas as pl
from jax.experimental.pallas import tpu as pltpu
from jax.experimental.pallas import tpu_sc as plsc

GRID_N = 1000
EMB_D = 16
BATCH_N = 16384

NUM_CORES = 2
NUM_SUBCORES = 16
LANES = 16
NUM_WORKERS = NUM_CORES * NUM_SUBCORES
ROWS_PER_WORKER = BATCH_N // NUM_WORKERS
IDX_CHUNK = 128
NUM_CHUNKS = ROWS_PER_WORKER // IDX_CHUNK
VECS_PER_CHUNK = IDX_CHUNK // LANES

_mesh = plsc.VectorSubcoreMesh(core_axis_name="c", subcore_axis_name="s")


@functools.partial(
    pl.kernel,
    out_type=jax.ShapeDtypeStruct((BATCH_N, EMB_D), jnp.float32),
    mesh=_mesh,
    compiler_params=pltpu.CompilerParams(use_tc_tiling_on_sc=False),
    scratch_types=[
        pltpu.VMEM((ROWS_PER_WORKER,), jnp.float32),
        pltpu.VMEM((ROWS_PER_WORKER,), jnp.float32),
        pltpu.VMEM((NUM_CHUNKS, IDX_CHUNK), jnp.int32),
        pltpu.VMEM((ROWS_PER_WORKER, EMB_D), jnp.float32),
        pltpu.SemaphoreType.DMA,
    ],
)
def _grid_lookup(xt_hbm, table_hbm, out_hbm, x0_v, x1_v, idx_v, rows_v, sem):
    wid = lax.axis_index("s") * NUM_CORES + lax.axis_index("c")
    base = wid * ROWS_PER_WORKER

    pltpu.sync_copy(xt_hbm.at[0, pl.ds(base, ROWS_PER_WORKER)], x0_v)
    pltpu.sync_copy(xt_hbm.at[1, pl.ds(base, ROWS_PER_WORKER)], x1_v)

    copies = []
    for t in range(NUM_CHUNKS):
        for v in range(VECS_PER_CHUNK):
            off = t * IDX_CHUNK + v * LANES
            x0 = x0_v[pl.ds(off, LANES)]
            x1 = x1_v[pl.ds(off, LANES)]
            i0 = (x0 * float(GRID_N)).astype(jnp.int32)
            i1 = (x1 * float(GRID_N)).astype(jnp.int32)
            idx_v.at[t][pl.ds(v * LANES, LANES)] = i0 * GRID_N + i1
        copies.append(
            pltpu.async_copy(
                table_hbm.at[idx_v.at[t]],
                rows_v.at[pl.ds(t * IDX_CHUNK, IDX_CHUNK)],
                sem,
            )
        )
    for c in copies:
        c.wait()

    pltpu.sync_copy(rows_v, out_hbm.at[pl.ds(base, ROWS_PER_WORKER)])


def kernel(x, table):
    return _grid_lookup(x.T, table)


# speedup vs baseline: 5.5598x; 5.5598x over previous
import functools

import jax
import jax.numpy as jnp
from jax import lax
from jax.experimental import pallas as pl
from jax.experimental.pallas import tpu as pltpu
from jax.experimental.pallas import tpu_sc as plsc

GRID_N = 1000
EMB_D = 16
BATCH_N = 16384

NUM_CORES = 2
NUM_SUBCORES = 16
LANES = 16
NUM_WORKERS = NUM_CORES * NUM_SUBCORES
PTS_PER_WORKER = BATCH_N // NUM_WORKERS
PTS_PER_CORE = BATCH_N // NUM_CORES
NUM_WAVES = PTS_PER_WORKER // LANES
RING = 2

_mesh = plsc.VectorSubcoreMesh(core_axis_name="c", subcore_axis_name="s")


@functools.partial(
    pl.kernel,
    out_type=jax.ShapeDtypeStruct((2, 8, BATCH_N), jnp.float32),
    mesh=_mesh,
    compiler_params=pltpu.CompilerParams(use_tc_tiling_on_sc=True),
    scratch_types=[
        pltpu.VMEM((PTS_PER_WORKER,), jnp.float32),
        pltpu.VMEM((PTS_PER_WORKER,), jnp.float32),
        pltpu.VMEM((RING * LANES, 2, 8, 128), jnp.float32),
        pltpu.VMEM((RING * LANES,), jnp.int32),
        pltpu.VMEM_SHARED((2, 8, PTS_PER_CORE), jnp.float32),
        pltpu.SemaphoreType.DMA,
        pltpu.SemaphoreType.DMA,
        pltpu.SemaphoreType.DMA,
    ],
)
def _grid_lookup(xt_hbm, t3_hbm, out_hbm, x0_v, x1_v, blk_v, lane_v, val_s,
                 sem_h0, sem_h1, sem_l):
    cid = lax.axis_index("c")
    sid = lax.axis_index("s")
    base = (cid * NUM_SUBCORES + sid) * PTS_PER_WORKER
    local = sid * PTS_PER_WORKER

    pltpu.sync_copy(xt_hbm.at[0, pl.ds(base, PTS_PER_WORKER)], x0_v)
    pltpu.sync_copy(xt_hbm.at[1, pl.ds(base, PTS_PER_WORKER)], x1_v)

    def issue_hbm(j, parity, sem):
        x0 = x0_v[pl.ds(j * LANES, LANES)]
        x1 = x1_v[pl.ds(j * LANES, LANES)]
        i0 = (x0 * float(GRID_N)).astype(jnp.int32)
        i1 = (x1 * float(GRID_N)).astype(jnp.int32)
        cells = i0 * GRID_N + i1
        slot = parity * LANES
        lane_v[pl.ds(slot, LANES)] = cells & 127
        c128 = cells & -128
        for k in range(LANES):
            off = pl.multiple_of(c128[k], 128)
            pltpu.async_copy(
                t3_hbm.at[:, :, pl.ds(off, 128)],
                blk_v.at[slot + k],
                sem,
            )

    def drain_extract(j, parity, sem):
        slot = parity * LANES
        for k in range(LANES):
            pltpu.make_async_copy(
                t3_hbm.at[:, :, pl.ds(0, 128)],
                blk_v.at[slot + k],
                sem,
            ).wait()
        lanes = lane_v[pl.ds(slot, LANES)]
        for k in range(LANES):
            pltpu.async_copy(
                blk_v.at[slot + k, :, :, pl.ds(lanes[k], 1)],
                val_s.at[:, :, pl.ds(local + j * LANES + k, 1)],
                sem_l,
            )
        pltpu.make_async_copy(
            t3_hbm.at[:, :, pl.ds(0, LANES)],
            val_s.at[:, :, pl.ds(local + j * LANES, LANES)],
            sem_l,
        ).wait()

    issue_hbm(0, 0, sem_h0)
    issue_hbm(1, 1, sem_h1)

    def body(i, carry):
        drain_extract(2 * i - 2, 0, sem_h0)
        issue_hbm(2 * i, 0, sem_h0)
        drain_extract(2 * i - 1, 1, sem_h1)
        issue_hbm(2 * i + 1, 1, sem_h1)
        return carry

    lax.fori_loop(1, NUM_WAVES // 2, body, 0)
    drain_extract(NUM_WAVES - 2, 0, sem_h0)
    drain_extract(NUM_WAVES - 1, 1, sem_h1)

    plsc.subcore_barrier()

    @pl.when(sid == 0)
    def _():
        pltpu.sync_copy(
            val_s, out_hbm.at[:, :, pl.ds(cid * PTS_PER_CORE, PTS_PER_CORE)]
        )


def kernel(x, table):
    t3 = table.T.reshape(2, 8, GRID_N * GRID_N)
    out3 = _grid_lookup(x.T, t3)
    return out3.reshape(EMB_D, BATCH_N).T
